# baseline (speedup 1.0000x reference)
import functools

import jax
import jax.numpy as jnp
from jax import lax
from jax.experimental import pallas as pl
from jax.experimental.pallas import tpu as pltpu

N = 50000
E = 800000
NUM_LINKS = 30
DT = 0.01
DZ = -0.0106666666666666

def _dzvec():
    lane = lax.broadcasted_iota(jnp.int32, (1, 6), 1)
    return jnp.where(lane == 2, DZ, 0.0).astype(jnp.float32)


def _small_mlp(v, params):
    (W1, b1), (W2, b2), (W3, b3) = params
    h = jnp.maximum(v * W1 + b1, 0.0)
    h = jnp.maximum(jnp.dot(h, W2, preferred_element_type=jnp.float32) + b2, 0.0)
    return jnp.dot(h, W3, preferred_element_type=jnp.float32) + b3


def _prep_body(x_ref, idf_ref, aW1, ab1, aW2, ab2, aW3, ab3, out_ref):
    x = x_ref[...]
    idf = idf_ref[...]
    alpha = _small_mlp(idf, ((aW1[...], ab1[...]), (aW2[...], ab2[...]),
                             (aW3[...], ab3[...])))
    xb = (x - idf * _dzvec()) * alpha
    out_ref[...] = jnp.concatenate(
        [xb, idf * (1.0 / 30.0), jnp.zeros_like(idf)], axis=1)


def _prep(x, idf, alpha_params, block=5000):
    flat = []
    for W, b in alpha_params:
        flat += [W, b.reshape(1, -1)]
    grid = N // block
    return pl.pallas_call(
        _prep_body,
        grid=(grid,),
        in_specs=[
            pl.BlockSpec((block, 6), lambda i: (i, 0)),
            pl.BlockSpec((block, 1), lambda i: (i, 0)),
        ] + [pl.BlockSpec(a.shape, lambda i: (0,) * a.ndim) for a in flat],
        out_specs=pl.BlockSpec((block, 8), lambda i: (i, 0)),
        out_shape=jax.ShapeDtypeStruct((N, 8), jnp.float32),
    )(x, idf, *flat)


def _edge_body(d_ref, W1, b1, W2, b2, W3, b3, W4, b4, out_ref):
    h = d_ref[...]
    h = jnp.maximum(jnp.dot(h, W1[...], preferred_element_type=jnp.float32) + b1[...], 0.0)
    h = jnp.maximum(jnp.dot(h, W2[...], preferred_element_type=jnp.float32) + b2[...], 0.0)
    h = jnp.maximum(jnp.dot(h, W3[...], preferred_element_type=jnp.float32) + b3[...], 0.0)
    h = jnp.dot(h, W4[...], preferred_element_type=jnp.float32) + b4[...]
    out_ref[0] = h[:, :32]
    out_ref[1] = h[:, 32:]


def _edge_mlp(diff, edge_params, block=8000):
    (W1, b1), (W2, b2), (W3, b3), (W4, b4) = edge_params
    W1p = jnp.zeros((8, 64), jnp.float32).at[:7].set(W1)
    flat = [W1p, b1.reshape(1, -1), W2, b2.reshape(1, -1),
            W3, b3.reshape(1, -1), W4, b4.reshape(1, -1)]
    grid = E // block
    return pl.pallas_call(
        _edge_body,
        grid=(grid,),
        in_specs=[pl.BlockSpec((block, 8), lambda i: (i, 0))]
        + [pl.BlockSpec(a.shape, lambda i: (0,) * a.ndim) for a in flat],
        out_specs=pl.BlockSpec((2, block, 32), lambda i: (0, i, 0)),
        out_shape=jax.ShapeDtypeStruct((2, E, 32), jnp.float32),
    )(diff, *flat)


def _node_body(xb_ref, u_ref, a2_ref, x_ref, idf_ref,
               W1a, W1u, W1lo, W1hi, b1, W2, b2, W3, b3, W4, b4,
               iW1, ib1, iW2, ib2, iW3, ib3, out_ref):
    xb = xb_ref[...]
    h = (jnp.dot(xb, W1a[...], preferred_element_type=jnp.float32)
         + jnp.dot(u_ref[...], W1u[...], preferred_element_type=jnp.float32)
         + jnp.dot(a2_ref[0], W1lo[...], preferred_element_type=jnp.float32)
         + jnp.dot(a2_ref[1], W1hi[...], preferred_element_type=jnp.float32)
         + b1[...])
    h = jnp.maximum(h, 0.0)
    h = jnp.maximum(jnp.dot(h, W2[...], preferred_element_type=jnp.float32) + b2[...], 0.0)
    h = jnp.maximum(jnp.dot(h, W3[...], preferred_element_type=jnp.float32) + b3[...], 0.0)
    dv = jnp.dot(h, W4[...], preferred_element_type=jnp.float32) + b4[...]
    ainv = _small_mlp(idf_ref[...], ((iW1[...], ib1[...]), (iW2[...], ib2[...]),
                                     (iW3[...], ib3[...])))
    dv = dv * ainv
    x = x_ref[...]
    v_new = x[:, 3:6] + dv
    x_new = x[:, 0:3] + v_new * DT
    out_ref[...] = jnp.concatenate([x_new, v_new], axis=1)


def _node_mlp(xb8, u, aggr2, x, idf, node_params, alpha_inv_params, block=5000):
    (W1, b1), (W2, b2), (W3, b3), (W4, b4) = node_params
    W1a = jnp.zeros((8, 128), jnp.float32).at[:7].set(W1[:7])
    W1u = W1[7:13]
    W1lo = W1[13:45]
    W1hi = W1[45:77]
    flat = [W1a, W1u, W1lo, W1hi, b1.reshape(1, -1), W2, b2.reshape(1, -1),
            W3, b3.reshape(1, -1), W4, b4.reshape(1, -1)]
    for W, b in alpha_inv_params:
        flat += [W, b.reshape(1, -1)]
    grid = N // block
    return pl.pallas_call(
        _node_body,
        grid=(grid,),
        in_specs=[
            pl.BlockSpec((block, 8), lambda i: (i, 0)),
            pl.BlockSpec((block, 6), lambda i: (i, 0)),
            pl.BlockSpec((2, block, 32), lambda i: (0, i, 0)),
            pl.BlockSpec((block, 6), lambda i: (i, 0)),
            pl.BlockSpec((block, 1), lambda i: (i, 0)),
        ] + [pl.BlockSpec(a.shape, lambda i: (0,) * a.ndim) for a in flat],
        out_specs=pl.BlockSpec((block, 6), lambda i: (i, 0)),
        out_shape=jax.ShapeDtypeStruct((N, 6), jnp.float32),
    )(xb8, u, aggr2, x, idf, *flat)


def kernel(x, ids, edge_index, u, alpha_params, alpha_inv_params, edge_params,
           node_params):
    idf = ids.astype(jnp.float32)
    sender = edge_index[0]
    receiver = edge_index[1]

    xb8 = _prep(x, idf, alpha_params)

    diff = xb8[receiver] - xb8[sender]

    eo2 = _edge_mlp(diff, edge_params)

    aggr2 = jnp.stack([
        jax.ops.segment_sum(eo2[0], receiver, num_segments=N),
        jax.ops.segment_sum(eo2[1], receiver, num_segments=N),
    ])

    return _node_mlp(xb8, u, aggr2, x, idf, node_params, alpha_inv_params)


# speedup vs baseline: 3.6951x; 3.6951x over previous
import functools

import jax
import jax.numpy as jnp
from jax import lax
from jax.experimental import pallas as pl
from jax.experimental.pallas import tpu as pltpu
from jax.experimental.pallas import tpu_sc as plsc

N = 50000
E = 800000
NUM_LINKS = 30
DT = 0.01
DZ = -0.0106666666666666

EP = 819200
NP = 50048
_NC, _NS = 2, 16
_NW = _NC * _NS

def _dzvec():
    lane = lax.broadcasted_iota(jnp.int32, (1, 6), 1)
    return jnp.where(lane == 2, DZ, 0.0).astype(jnp.float32)


def _small_mlp(v, params):
    (W1, b1), (W2, b2), (W3, b3) = params
    h = jnp.maximum(v * W1 + b1, 0.0)
    h = jnp.maximum(jnp.dot(h, W2, preferred_element_type=jnp.float32) + b2, 0.0)
    return jnp.dot(h, W3, preferred_element_type=jnp.float32) + b3


def _prep_body(x_ref, idf_ref, aW1, ab1, aW2, ab2, aW3, ab3, out_ref):
    x = x_ref[...]
    idf = idf_ref[...]
    alpha = _small_mlp(idf, ((aW1[...], ab1[...]), (aW2[...], ab2[...]),
                             (aW3[...], ab3[...])))
    xb = (x - idf * _dzvec()) * alpha
    out_ref[...] = jnp.concatenate(
        [xb, idf * (1.0 / 30.0), jnp.zeros_like(idf)], axis=1)


def _prep(x, idf, alpha_params, block=5000):
    flat = []
    for W, b in alpha_params:
        flat += [W, b.reshape(1, -1)]
    grid = N // block
    return pl.pallas_call(
        _prep_body,
        grid=(grid,),
        in_specs=[
            pl.BlockSpec((block, 6), lambda i: (i, 0)),
            pl.BlockSpec((block, 1), lambda i: (i, 0)),
        ] + [pl.BlockSpec(a.shape, lambda i: (0,) * a.ndim) for a in flat],
        out_specs=pl.BlockSpec((block, 8), lambda i: (i, 0)),
        out_shape=jax.ShapeDtypeStruct((N, 8), jnp.float32),
    )(x, idf, *flat)


_EPW = EP // _NW
_GCH = 1024
_GNCH = _EPW // _GCH


def _sc_gather(xbp, send_p, recv_p):
    mesh = plsc.VectorSubcoreMesh(core_axis_name="c", subcore_axis_name="s")

    @functools.partial(
        pl.kernel,
        out_type=jax.ShapeDtypeStruct((2, EP, 8), jnp.float32),
        mesh=mesh,
        scratch_types=[
            pltpu.VMEM((_GCH,), jnp.int32),
            pltpu.VMEM((_GCH,), jnp.int32),
            pltpu.VMEM((_GCH, 8), jnp.float32),
            pltpu.VMEM((_GCH, 8), jnp.float32),
            pltpu.SemaphoreType.DMA,
        ],
        compiler_params=pltpu.CompilerParams(use_tc_tiling_on_sc=False),
    )
    def k(xb_hbm, send_hbm, recv_hbm, out_hbm, idx_r, idx_s, buf_r, buf_s, sem):
        wid = lax.axis_index("s") * _NC + lax.axis_index("c")
        base0 = wid * _EPW

        def chunk(i, carry):
            base = base0 + i * _GCH
            pltpu.sync_copy(recv_hbm.at[pl.ds(base, _GCH)], idx_r)
            pltpu.sync_copy(send_hbm.at[pl.ds(base, _GCH)], idx_s)
            cps = []
            for j in range(_GCH // 128):
                sl = pl.ds(j * 128, 128)
                cps.append(pltpu.async_copy(xb_hbm.at[idx_r.at[sl]],
                                            buf_r.at[sl], sem))
                cps.append(pltpu.async_copy(xb_hbm.at[idx_s.at[sl]],
                                            buf_s.at[sl], sem))
            for cp in cps:
                cp.wait()
            pltpu.sync_copy(buf_r, out_hbm.at[0, pl.ds(base, _GCH)])
            pltpu.sync_copy(buf_s, out_hbm.at[1, pl.ds(base, _GCH)])
            return carry

        lax.fori_loop(0, _GNCH, chunk, 0)

    return k(xbp, send_p, recv_p)


_SPAN = NP // _NS
_EPT = EP // _NS
_SCH = 640
_SNCH = _EPT // _SCH


def _sc_scatter(eo2, recv3d, zspan):
    mesh = plsc.VectorSubcoreMesh(core_axis_name="c", subcore_axis_name="s")

    @functools.partial(
        pl.kernel,
        out_type=jax.ShapeDtypeStruct((2, NP, 32), jnp.float32),
        mesh=mesh,
        scratch_types=[
            pltpu.VMEM_SHARED((NP, 32), jnp.float32),
            pltpu.VMEM((_SCH // 128, 128), jnp.int32),
            pltpu.VMEM((_SCH, 32), jnp.float32),
            pltpu.SemaphoreType.DMA,
        ],
        compiler_params=pltpu.CompilerParams(use_tc_tiling_on_sc=False),
    )
    def k(eo_hbm, recv_hbm, z_hbm, out_hbm, acc, idx2, rows, sem):
        c = lax.axis_index("c")
        s = lax.axis_index("s")
        span = pl.ds(s * _SPAN, _SPAN)
        pltpu.sync_copy(z_hbm, acc.at[span])
        plsc.subcore_barrier()

        def chunk(i, carry):
            ci = s * _SNCH + i
            pltpu.sync_copy(recv_hbm.at[ci], idx2)
            pltpu.sync_copy(eo_hbm.at[c, pl.ds(ci * _SCH, _SCH)], rows)
            for j in range(_SCH // 128):
                pltpu.async_copy(rows.at[pl.ds(j * 128, 128)],
                                 acc.at[idx2.at[j]], sem, add=True).wait()
            return carry

        lax.fori_loop(0, _SNCH, chunk, 0)
        plsc.subcore_barrier()
        pltpu.sync_copy(acc.at[span], out_hbm.at[c, span])

    return k(eo2, recv3d, zspan)


def _edge_body(d_ref, W1, b1, W2, b2, W3, b3, W4, b4, out_ref):
    h = d_ref[0] - d_ref[1]
    h = jnp.maximum(jnp.dot(h, W1[...], preferred_element_type=jnp.float32) + b1[...], 0.0)
    h = jnp.maximum(jnp.dot(h, W2[...], preferred_element_type=jnp.float32) + b2[...], 0.0)
    h = jnp.maximum(jnp.dot(h, W3[...], preferred_element_type=jnp.float32) + b3[...], 0.0)
    h = jnp.dot(h, W4[...], preferred_element_type=jnp.float32) + b4[...]
    out_ref[0] = h[:, :32]
    out_ref[1] = h[:, 32:]


def _edge_mlp(g2, edge_params, block=8192):
    (W1, b1), (W2, b2), (W3, b3), (W4, b4) = edge_params
    W1p = jnp.zeros((8, 64), jnp.float32).at[:7].set(W1)
    flat = [W1p, b1.reshape(1, -1), W2, b2.reshape(1, -1),
            W3, b3.reshape(1, -1), W4, b4.reshape(1, -1)]
    grid = EP // block
    return pl.pallas_call(
        _edge_body,
        grid=(grid,),
        in_specs=[pl.BlockSpec((2, block, 8), lambda i: (0, i, 0))]
        + [pl.BlockSpec(a.shape, lambda i: (0,) * a.ndim) for a in flat],
        out_specs=pl.BlockSpec((2, block, 32), lambda i: (0, i, 0)),
        out_shape=jax.ShapeDtypeStruct((2, EP, 32), jnp.float32),
    )(g2, *flat)


def _node_body(xb_ref, u_ref, a2_ref, x_ref, idf_ref,
               W1a, W1u, W1lo, W1hi, b1, W2, b2, W3, b3, W4, b4,
               iW1, ib1, iW2, ib2, iW3, ib3, out_ref):
    xb = xb_ref[...]
    h = (jnp.dot(xb, W1a[...], preferred_element_type=jnp.float32)
         + jnp.dot(u_ref[...], W1u[...], preferred_element_type=jnp.float32)
         + jnp.dot(a2_ref[0], W1lo[...], preferred_element_type=jnp.float32)
         + jnp.dot(a2_ref[1], W1hi[...], preferred_element_type=jnp.float32)
         + b1[...])
    h = jnp.maximum(h, 0.0)
    h = jnp.maximum(jnp.dot(h, W2[...], preferred_element_type=jnp.float32) + b2[...], 0.0)
    h = jnp.maximum(jnp.dot(h, W3[...], preferred_element_type=jnp.float32) + b3[...], 0.0)
    dv = jnp.dot(h, W4[...], preferred_element_type=jnp.float32) + b4[...]
    ainv = _small_mlp(idf_ref[...], ((iW1[...], ib1[...]), (iW2[...], ib2[...]),
                                     (iW3[...], ib3[...])))
    dv = dv * ainv
    x = x_ref[...]
    v_new = x[:, 3:6] + dv
    x_new = x[:, 0:3] + v_new * DT
    out_ref[...] = jnp.concatenate([x_new, v_new], axis=1)


def _node_mlp(xb8, u, aggr2, x, idf, node_params, alpha_inv_params, block=5000):
    (W1, b1), (W2, b2), (W3, b3), (W4, b4) = node_params
    W1a = jnp.zeros((8, 128), jnp.float32).at[:7].set(W1[:7])
    W1u = W1[7:13]
    W1lo = W1[13:45]
    W1hi = W1[45:77]
    flat = [W1a, W1u, W1lo, W1hi, b1.reshape(1, -1), W2, b2.reshape(1, -1),
            W3, b3.reshape(1, -1), W4, b4.reshape(1, -1)]
    for W, b in alpha_inv_params:
        flat += [W, b.reshape(1, -1)]
    grid = N // block
    return pl.pallas_call(
        _node_body,
        grid=(grid,),
        in_specs=[
            pl.BlockSpec((block, 8), lambda i: (i, 0)),
            pl.BlockSpec((block, 6), lambda i: (i, 0)),
            pl.BlockSpec((2, block, 32), lambda i: (0, i, 0)),
            pl.BlockSpec((block, 6), lambda i: (i, 0)),
            pl.BlockSpec((block, 1), lambda i: (i, 0)),
        ] + [pl.BlockSpec(a.shape, lambda i: (0,) * a.ndim) for a in flat],
        out_specs=pl.BlockSpec((block, 6), lambda i: (i, 0)),
        out_shape=jax.ShapeDtypeStruct((N, 6), jnp.float32),
    )(xb8, u, aggr2, x, idf, *flat)


def kernel(x, ids, edge_index, u, alpha_params, alpha_inv_params, edge_params,
           node_params):
    idf = ids.astype(jnp.float32)
    sender = edge_index[0]
    receiver = edge_index[1]

    xb8 = _prep(x, idf, alpha_params)
    xbp = jnp.zeros((NP, 8), jnp.float32).at[:N].set(xb8)

    send_p = jnp.concatenate([sender, jnp.zeros((EP - E,), jnp.int32)])
    recv_p = jnp.concatenate([receiver, jnp.full((EP - E,), N, jnp.int32)])
    recv3d = recv_p.reshape(EP // _SCH, _SCH // 128, 128)

    g2 = _sc_gather(xbp, send_p, recv_p)

    eo2 = _edge_mlp(g2, edge_params)

    zspan = jnp.zeros((_SPAN, 32), jnp.float32)
    aggr2p = _sc_scatter(eo2, recv3d, zspan)

    return _node_mlp(xbp, u, aggr2p, x, idf, node_params, alpha_inv_params)


# speedup vs baseline: 7.7712x; 2.1031x over previous
import functools

import numpy as np
import jax
import jax.numpy as jnp
from jax import lax
from jax.experimental import pallas as pl
from jax.experimental.pallas import tpu as pltpu
from jax.experimental.pallas import tpu_sc as plsc

N = 50000
E = 800000
NUM_LINKS = 30
DT = 0.01
DZ = -0.0106666666666666

EP = 819200
NP = 50048
_NC, _NS = 2, 16
_NW = _NC * _NS

def _dzvec():
    lane = lax.broadcasted_iota(jnp.int32, (1, 6), 1)
    return jnp.where(lane == 2, DZ, 0.0).astype(jnp.float32)


def _small_mlp(v, params):
    (W1, b1), (W2, b2), (W3, b3) = params
    h = jnp.maximum(v * W1 + b1, 0.0)
    h = jnp.maximum(jnp.dot(h, W2, preferred_element_type=jnp.float32) + b2, 0.0)
    return jnp.dot(h, W3, preferred_element_type=jnp.float32) + b3


def _prep_body(x_ref, idf_ref, aW1, ab1, aW2, ab2, aW3, ab3, out_ref):
    x = x_ref[...]
    idf = idf_ref[...]
    alpha = _small_mlp(idf, ((aW1[...], ab1[...]), (aW2[...], ab2[...]),
                             (aW3[...], ab3[...])))
    xb = (x - idf * _dzvec()) * alpha
    out_ref[...] = jnp.concatenate(
        [xb, idf * (1.0 / 30.0), jnp.zeros_like(idf)], axis=1)


def _prep(x, idf, alpha_params, block=5000):
    flat = []
    for W, b in alpha_params:
        flat += [W, b.reshape(1, -1)]
    grid = N // block
    return pl.pallas_call(
        _prep_body,
        grid=(grid,),
        in_specs=[
            pl.BlockSpec((block, 6), lambda i: (i, 0)),
            pl.BlockSpec((block, 1), lambda i: (i, 0)),
        ] + [pl.BlockSpec(a.shape, lambda i: (0,) * a.ndim) for a in flat],
        out_specs=pl.BlockSpec((block, 8), lambda i: (i, 0)),
        out_shape=jax.ShapeDtypeStruct((N, 8), jnp.float32),
    )(x, idf, *flat)


_EPW = EP // _NW
_GCH = 1024
_GNCH = _EPW // _GCH


def _sc_gather(xbp, send_p, recv_p):
    mesh = plsc.VectorSubcoreMesh(core_axis_name="c", subcore_axis_name="s")

    @functools.partial(
        pl.kernel,
        out_type=jax.ShapeDtypeStruct((2, EP, 8), jnp.float32),
        mesh=mesh,
        scratch_types=[
            pltpu.VMEM((_GCH,), jnp.int32),
            pltpu.VMEM((_GCH,), jnp.int32),
            pltpu.VMEM((_GCH, 8), jnp.float32),
            pltpu.VMEM((_GCH, 8), jnp.float32),
            pltpu.SemaphoreType.DMA,
        ],
        compiler_params=pltpu.CompilerParams(use_tc_tiling_on_sc=False),
    )
    def k(xb_hbm, send_hbm, recv_hbm, out_hbm, idx_r, idx_s, buf_r, buf_s, sem):
        wid = lax.axis_index("s") * _NC + lax.axis_index("c")
        base0 = wid * _EPW

        def chunk(i, carry):
            base = base0 + i * _GCH
            pltpu.sync_copy(recv_hbm.at[pl.ds(base, _GCH)], idx_r)
            pltpu.sync_copy(send_hbm.at[pl.ds(base, _GCH)], idx_s)
            cps = []
            for j in range(_GCH // 128):
                sl = pl.ds(j * 128, 128)
                cps.append(pltpu.async_copy(xb_hbm.at[idx_r.at[sl]],
                                            buf_r.at[sl], sem))
                cps.append(pltpu.async_copy(xb_hbm.at[idx_s.at[sl]],
                                            buf_s.at[sl], sem))
            for cp in cps:
                cp.wait()
            pltpu.sync_copy(buf_r, out_hbm.at[0, pl.ds(base, _GCH)])
            pltpu.sync_copy(buf_s, out_hbm.at[1, pl.ds(base, _GCH)])
            return carry

        lax.fori_loop(0, _GNCH, chunk, 0)

    return k(xbp, send_p, recv_p)


_SPAN = NP // _NS
_EPT = EP // _NS
_SCH = 640
_SNCH = _EPT // _SCH


def _sc_scatter(eo2, recv3d, zspan):
    mesh = plsc.VectorSubcoreMesh(core_axis_name="c", subcore_axis_name="s")

    @functools.partial(
        pl.kernel,
        out_type=jax.ShapeDtypeStruct((2, NP, 32), jnp.float32),
        mesh=mesh,
        scratch_types=[
            pltpu.VMEM_SHARED((NP, 32), jnp.float32),
            pltpu.VMEM((_SCH // 128, 128), jnp.int32),
            pltpu.VMEM((_SCH, 32), jnp.float32),
            pltpu.SemaphoreType.DMA,
        ],
        compiler_params=pltpu.CompilerParams(use_tc_tiling_on_sc=False),
    )
    def k(eo_hbm, recv_hbm, z_hbm, out_hbm, acc, idx2, rows, sem):
        c = lax.axis_index("c")
        s = lax.axis_index("s")
        span = pl.ds(s * _SPAN, _SPAN)
        pltpu.sync_copy(z_hbm, acc.at[span])
        plsc.subcore_barrier()

        def chunk(i, carry):
            ci = s * _SNCH + i
            pltpu.sync_copy(recv_hbm.at[ci], idx2)
            pltpu.sync_copy(eo_hbm.at[c, pl.ds(ci * _SCH, _SCH)], rows)
            for j in range(_SCH // 128):
                pltpu.async_copy(rows.at[pl.ds(j * 128, 128)],
                                 acc.at[idx2.at[j]], sem, add=True).wait()
            return carry

        lax.fori_loop(0, _SNCH, chunk, 0)
        plsc.subcore_barrier()
        pltpu.sync_copy(acc.at[span], out_hbm.at[c, span])

    return k(eo2, recv3d, zspan)


def _bdiag(W, k):
    return jax.scipy.linalg.block_diag(*([W] * k))


def _edge_body(d_ref, W1, b1, W2, b2, W3, b3, W4, b4, out_ref):
    h = d_ref[0] - d_ref[1]
    h = jnp.maximum(jnp.dot(h, W1[...], preferred_element_type=jnp.float32) + b1[...], 0.0)
    h = jnp.maximum(jnp.dot(h, W2[...], preferred_element_type=jnp.float32) + b2[...], 0.0)
    h = jnp.maximum(jnp.dot(h, W3[...], preferred_element_type=jnp.float32) + b3[...], 0.0)
    h = jnp.dot(h, W4[...], preferred_element_type=jnp.float32) + b4[...]
    out_ref[0] = h[:, :128]
    out_ref[1] = h[:, 128:]


def _edge_mlp(g2, edge_params, block=2048):
    (W1, b1), (W2, b2), (W3, b3), (W4, b4) = edge_params
    W1p = jnp.zeros((8, 64), jnp.float32).at[:7].set(W1)
    old = np.empty((256,), np.int64)
    for p in range(256):
        if p < 128:
            e, j = p // 32, p % 32
            old[p] = 64 * e + j
        else:
            e, j = (p - 128) // 32, (p - 128) % 32
            old[p] = 64 * e + 32 + j
    W4pd = _bdiag(W4, 4)[:, old]
    b4pd = jnp.tile(b4, 4)[old]
    flat = [_bdiag(W1p, 4), jnp.tile(b1, 4).reshape(1, -1),
            _bdiag(W2, 4), jnp.tile(b2, 4).reshape(1, -1),
            _bdiag(W3, 4), jnp.tile(b3, 4).reshape(1, -1),
            W4pd, b4pd.reshape(1, -1)]
    g2p = g2.reshape(2, EP // 4, 32)
    grid = (EP // 4) // block
    out = pl.pallas_call(
        _edge_body,
        grid=(grid,),
        in_specs=[pl.BlockSpec((2, block, 32), lambda i: (0, i, 0))]
        + [pl.BlockSpec(a.shape, lambda i: (0,) * a.ndim) for a in flat],
        out_specs=pl.BlockSpec((2, block, 128), lambda i: (0, i, 0)),
        out_shape=jax.ShapeDtypeStruct((2, EP // 4, 128), jnp.float32),
    )(g2p, *flat)
    return out.reshape(2, EP, 32)


def _node_body(xb_ref, u_ref, a2_ref, x_ref, idf_ref,
               W1a, W1u, W1lo, W1hi, b1, W2, b2, W3, b3, W4, b4,
               iW1, ib1, iW2, ib2, iW3, ib3, out_ref):
    h = (jnp.dot(xb_ref[...], W1a[...], preferred_element_type=jnp.float32)
         + jnp.dot(u_ref[...], W1u[...], preferred_element_type=jnp.float32)
         + jnp.dot(a2_ref[0], W1lo[...], preferred_element_type=jnp.float32)
         + jnp.dot(a2_ref[1], W1hi[...], preferred_element_type=jnp.float32)
         + b1[...])
    h = jnp.maximum(h, 0.0)
    h = jnp.maximum(jnp.dot(h, W2[...], preferred_element_type=jnp.float32) + b2[...], 0.0)
    h = jnp.maximum(jnp.dot(h, W3[...], preferred_element_type=jnp.float32) + b3[...], 0.0)
    dv = jnp.dot(h, W4[...], preferred_element_type=jnp.float32) + b4[...]
    ip = ((iW1[...], ib1[...]), (iW2[...], ib2[...]), (iW3[...], ib3[...]))
    idf = idf_ref[...]
    ia = _small_mlp(idf[:, 0:1], ip)
    ib = _small_mlp(idf[:, 1:2], ip)
    dv = dv * jnp.concatenate([ia, ia, ia, ib, ib, ib], axis=1)
    x = x_ref[...]
    v0 = x[:, 3:6] + dv[:, 0:3]
    x0 = x[:, 0:3] + v0 * DT
    v1 = x[:, 9:12] + dv[:, 3:6]
    x1 = x[:, 6:9] + v1 * DT
    out_ref[...] = jnp.concatenate([x0, v0, x1, v1], axis=1)


def _node_mlp(xbp, u, aggr2p, x, idf, node_params, alpha_inv_params, block=5000):
    (W1, b1), (W2, b2), (W3, b3), (W4, b4) = node_params
    W1a = jnp.zeros((8, 128), jnp.float32).at[:7].set(W1[:7])
    W1u = W1[7:13]
    W1lo = W1[13:45]
    W1hi = W1[45:77]
    flat = [_bdiag(W1a, 2), _bdiag(W1u, 2), _bdiag(W1lo, 2), _bdiag(W1hi, 2),
            jnp.tile(b1, 2).reshape(1, -1), _bdiag(W2, 2),
            jnp.tile(b2, 2).reshape(1, -1), _bdiag(W3, 2),
            jnp.tile(b3, 2).reshape(1, -1), _bdiag(W4, 2),
            jnp.tile(b4, 2).reshape(1, -1)]
    for W, b in alpha_inv_params:
        flat += [W, b.reshape(1, -1)]
    grid = (N // 2) // block
    out = pl.pallas_call(
        _node_body,
        grid=(grid,),
        in_specs=[
            pl.BlockSpec((block, 16), lambda i: (i, 0)),
            pl.BlockSpec((block, 12), lambda i: (i, 0)),
            pl.BlockSpec((2, block, 64), lambda i: (0, i, 0)),
            pl.BlockSpec((block, 12), lambda i: (i, 0)),
            pl.BlockSpec((block, 2), lambda i: (i, 0)),
        ] + [pl.BlockSpec(a.shape, lambda i: (0,) * a.ndim) for a in flat],
        out_specs=pl.BlockSpec((block, 12), lambda i: (i, 0)),
        out_shape=jax.ShapeDtypeStruct((N // 2, 12), jnp.float32),
    )(xbp.reshape(NP // 2, 16), u.reshape(N // 2, 12),
      aggr2p.reshape(2, NP // 2, 64), x.reshape(N // 2, 12),
      idf.reshape(N // 2, 2), *flat)
    return out.reshape(N, 6)


def kernel(x, ids, edge_index, u, alpha_params, alpha_inv_params, edge_params,
           node_params):
    idf = ids.astype(jnp.float32)
    sender = edge_index[0]
    receiver = edge_index[1]

    xb8 = _prep(x, idf, alpha_params)
    xbp = jnp.zeros((NP, 8), jnp.float32).at[:N].set(xb8)

    send_p = jnp.concatenate([sender, jnp.zeros((EP - E,), jnp.int32)])
    recv_p = jnp.concatenate([receiver, jnp.full((EP - E,), N, jnp.int32)])
    recv3d = recv_p.reshape(EP // _SCH, _SCH // 128, 128)

    g2 = _sc_gather(xbp, send_p, recv_p)

    eo2 = _edge_mlp(g2, edge_params)

    zspan = jnp.zeros((_SPAN, 32), jnp.float32)
    aggr2p = _sc_scatter(eo2, recv3d, zspan)

    return _node_mlp(xbp, u, aggr2p, x, idf, node_params, alpha_inv_params)


# speedup vs baseline: 8.2212x; 1.0579x over previous
import functools

import numpy as np
import jax
import jax.numpy as jnp
from jax import lax
from jax.experimental import pallas as pl
from jax.experimental.pallas import tpu as pltpu
from jax.experimental.pallas import tpu_sc as plsc

N = 50000
E = 800000
NUM_LINKS = 30
DT = 0.01
DZ = -0.0106666666666666

EP = 819200
NP = 50048
_NC, _NS = 2, 16
_NW = _NC * _NS

def _dzvec():
    lane = lax.broadcasted_iota(jnp.int32, (1, 6), 1)
    return jnp.where(lane == 2, DZ, 0.0).astype(jnp.float32)


def _small_mlp(v, params):
    (W1, b1), (W2, b2), (W3, b3) = params
    h = jnp.maximum(v * W1 + b1, 0.0)
    h = jnp.maximum(jnp.dot(h, W2, preferred_element_type=jnp.float32) + b2, 0.0)
    return jnp.dot(h, W3, preferred_element_type=jnp.float32) + b3


def _prep_body(x_ref, idf_ref, aW1, ab1, aW2, ab2, aW3, ab3, out_ref):
    x = x_ref[...]
    idf = idf_ref[...]
    alpha = _small_mlp(idf, ((aW1[...], ab1[...]), (aW2[...], ab2[...]),
                             (aW3[...], ab3[...])))
    xb = (x - idf * _dzvec()) * alpha
    out_ref[...] = jnp.concatenate(
        [xb, idf * (1.0 / 30.0), jnp.zeros_like(idf)], axis=1)


def _prep(x, idf, alpha_params, block=6256):
    flat = []
    for W, b in alpha_params:
        flat += [W, b.reshape(1, -1)]
    grid = NP // block
    return pl.pallas_call(
        _prep_body,
        grid=(grid,),
        in_specs=[
            pl.BlockSpec((block, 6), lambda i: (i, 0)),
            pl.BlockSpec((block, 1), lambda i: (i, 0)),
        ] + [pl.BlockSpec(a.shape, lambda i: (0,) * a.ndim) for a in flat],
        out_specs=pl.BlockSpec((block, 8), lambda i: (i, 0)),
        out_shape=jax.ShapeDtypeStruct((NP, 8), jnp.float32),
    )(x, idf, *flat)


_EPW = EP // _NW
_GCH = 1024
_GNCH = _EPW // _GCH


def _sc_gather(xbp, send_p, recv_p):
    mesh = plsc.VectorSubcoreMesh(core_axis_name="c", subcore_axis_name="s")

    @functools.partial(
        pl.kernel,
        out_type=jax.ShapeDtypeStruct((2, EP, 8), jnp.float32),
        mesh=mesh,
        scratch_types=[
            pltpu.VMEM((_GCH,), jnp.int32),
            pltpu.VMEM((_GCH,), jnp.int32),
            pltpu.VMEM((_GCH, 8), jnp.float32),
            pltpu.VMEM((_GCH, 8), jnp.float32),
            pltpu.SemaphoreType.DMA,
        ],
        compiler_params=pltpu.CompilerParams(use_tc_tiling_on_sc=False),
    )
    def k(xb_hbm, send_hbm, recv_hbm, out_hbm, idx_r, idx_s, buf_r, buf_s, sem):
        wid = lax.axis_index("s") * _NC + lax.axis_index("c")
        base0 = wid * _EPW

        def chunk(i, carry):
            base = base0 + i * _GCH
            pltpu.sync_copy(recv_hbm.at[pl.ds(base, _GCH)], idx_r)
            pltpu.sync_copy(send_hbm.at[pl.ds(base, _GCH)], idx_s)
            cps = []
            for j in range(_GCH // 128):
                sl = pl.ds(j * 128, 128)
                cps.append(pltpu.async_copy(xb_hbm.at[idx_r.at[sl]],
                                            buf_r.at[sl], sem))
                cps.append(pltpu.async_copy(xb_hbm.at[idx_s.at[sl]],
                                            buf_s.at[sl], sem))
            for cp in cps:
                cp.wait()
            pltpu.sync_copy(buf_r, out_hbm.at[0, pl.ds(base, _GCH)])
            pltpu.sync_copy(buf_s, out_hbm.at[1, pl.ds(base, _GCH)])
            return carry

        lax.fori_loop(0, _GNCH, chunk, 0)

    return k(xbp, send_p, recv_p)


_SPAN = NP // _NS
_EPT = EP // _NS
_SCH = 640
_SNCH = _EPT // _SCH


def _sc_scatter(eo2, recv3d, zspan):
    mesh = plsc.VectorSubcoreMesh(core_axis_name="c", subcore_axis_name="s")

    @functools.partial(
        pl.kernel,
        out_type=jax.ShapeDtypeStruct((2, NP, 32), jnp.float32),
        mesh=mesh,
        scratch_types=[
            pltpu.VMEM_SHARED((NP, 32), jnp.float32),
            pltpu.VMEM((_SCH // 128, 128), jnp.int32),
            pltpu.VMEM((_SCH, 32), jnp.float32),
            pltpu.SemaphoreType.DMA,
        ],
        compiler_params=pltpu.CompilerParams(use_tc_tiling_on_sc=False),
    )
    def k(eo_hbm, recv_hbm, z_hbm, out_hbm, acc, idx2, rows, sem):
        c = lax.axis_index("c")
        s = lax.axis_index("s")
        span = pl.ds(s * _SPAN, _SPAN)
        pltpu.sync_copy(z_hbm, acc.at[span])
        plsc.subcore_barrier()

        def chunk(i, carry):
            ci = s * _SNCH + i
            pltpu.sync_copy(recv_hbm.at[ci], idx2)
            pltpu.sync_copy(eo_hbm.at[c, pl.ds(ci * _SCH, _SCH)], rows)
            for j in range(_SCH // 128):
                pltpu.async_copy(rows.at[pl.ds(j * 128, 128)],
                                 acc.at[idx2.at[j]], sem, add=True).wait()
            return carry

        lax.fori_loop(0, _SNCH, chunk, 0)
        plsc.subcore_barrier()
        pltpu.sync_copy(acc.at[span], out_hbm.at[c, span])

    return k(eo2, recv3d, zspan)


def _bdiag(W, k):
    return jax.scipy.linalg.block_diag(*([W] * k))


def _edge_body(d_ref, W1, b1, W2, b2, W3, b3, W4, b4, out_ref):
    d = d_ref[0] - d_ref[1]
    h = jnp.concatenate(
        [d[:, 0:32], d[:, 32:64], d[:, 64:96], d[:, 96:128]], axis=0)
    h = jnp.maximum(jnp.dot(h, W1[...], preferred_element_type=jnp.float32) + b1[...], 0.0)
    h = jnp.maximum(jnp.dot(h, W2[...], preferred_element_type=jnp.float32) + b2[...], 0.0)
    h = jnp.maximum(jnp.dot(h, W3[...], preferred_element_type=jnp.float32) + b3[...], 0.0)
    h = jnp.dot(h, W4[...], preferred_element_type=jnp.float32) + b4[...]
    out_ref[0] = h[:, :128]
    out_ref[1] = h[:, 128:]


_EB16 = 512


def _edge_mlp(g2, edge_params, block=_EB16):
    (W1, b1), (W2, b2), (W3, b3), (W4, b4) = edge_params
    W1p = jnp.zeros((8, 64), jnp.float32).at[:7].set(W1)
    old = np.empty((256,), np.int64)
    for p in range(256):
        if p < 128:
            e, j = p // 32, p % 32
            old[p] = 64 * e + j
        else:
            e, j = (p - 128) // 32, (p - 128) % 32
            old[p] = 64 * e + 32 + j
    W4pd = _bdiag(W4, 4)[:, old]
    b4pd = jnp.tile(b4, 4)[old]
    flat = [_bdiag(W1p, 4), jnp.tile(b1, 4).reshape(1, -1),
            _bdiag(W2, 4), jnp.tile(b2, 4).reshape(1, -1),
            _bdiag(W3, 4), jnp.tile(b3, 4).reshape(1, -1),
            W4pd, b4pd.reshape(1, -1)]
    g128 = g2.reshape(2, EP // 16, 128)
    grid = (EP // 16) // block
    out = pl.pallas_call(
        _edge_body,
        grid=(grid,),
        in_specs=[pl.BlockSpec((2, block, 128), lambda i: (0, i, 0))]
        + [pl.BlockSpec(a.shape, lambda i: (0,) * a.ndim) for a in flat],
        out_specs=pl.BlockSpec((2, block * 4, 128), lambda i: (0, i, 0)),
        out_shape=jax.ShapeDtypeStruct((2, EP // 4, 128), jnp.float32),
    )(g128, *flat)
    return out.reshape(2, EP, 32)


def _node_body(xb_ref, u_ref, a2_ref, x_ref, idf_ref,
               W1a, W1u, W1lo, W1hi, b1, W2, b2, W3, b3, W4, b4,
               iW1, ib1, iW2, ib2, iW3, ib3, out_ref):
    h = (jnp.dot(xb_ref[...], W1a[...], preferred_element_type=jnp.float32)
         + jnp.dot(u_ref[...], W1u[...], preferred_element_type=jnp.float32)
         + jnp.dot(a2_ref[0], W1lo[...], preferred_element_type=jnp.float32)
         + jnp.dot(a2_ref[1], W1hi[...], preferred_element_type=jnp.float32)
         + b1[...])
    h = jnp.maximum(h, 0.0)
    h = jnp.maximum(jnp.dot(h, W2[...], preferred_element_type=jnp.float32) + b2[...], 0.0)
    h = jnp.maximum(jnp.dot(h, W3[...], preferred_element_type=jnp.float32) + b3[...], 0.0)
    dv = jnp.dot(h, W4[...], preferred_element_type=jnp.float32) + b4[...]
    ip = ((iW1[...], ib1[...]), (iW2[...], ib2[...]), (iW3[...], ib3[...]))
    idf = idf_ref[...]
    ia = _small_mlp(idf[:, 0:1], ip)
    ib = _small_mlp(idf[:, 1:2], ip)
    dv = dv * jnp.concatenate([ia, ia, ia, ib, ib, ib], axis=1)
    x = x_ref[...]
    v0 = x[:, 3:6] + dv[:, 0:3]
    x0 = x[:, 0:3] + v0 * DT
    v1 = x[:, 9:12] + dv[:, 3:6]
    x1 = x[:, 6:9] + v1 * DT
    out_ref[...] = jnp.concatenate([x0, v0, x1, v1], axis=1)


def _node_mlp(xbp, u, aggr2p, x, idf, node_params, alpha_inv_params, block=5000):
    (W1, b1), (W2, b2), (W3, b3), (W4, b4) = node_params
    W1a = jnp.zeros((8, 128), jnp.float32).at[:7].set(W1[:7])
    W1u = W1[7:13]
    W1lo = W1[13:45]
    W1hi = W1[45:77]
    flat = [_bdiag(W1a, 2), _bdiag(W1u, 2), _bdiag(W1lo, 2), _bdiag(W1hi, 2),
            jnp.tile(b1, 2).reshape(1, -1), _bdiag(W2, 2),
            jnp.tile(b2, 2).reshape(1, -1), _bdiag(W3, 2),
            jnp.tile(b3, 2).reshape(1, -1), _bdiag(W4, 2),
            jnp.tile(b4, 2).reshape(1, -1)]
    for W, b in alpha_inv_params:
        flat += [W, b.reshape(1, -1)]
    grid = (N // 2) // block
    out = pl.pallas_call(
        _node_body,
        grid=(grid,),
        in_specs=[
            pl.BlockSpec((block, 16), lambda i: (i, 0)),
            pl.BlockSpec((block, 12), lambda i: (i, 0)),
            pl.BlockSpec((2, block, 64), lambda i: (0, i, 0)),
            pl.BlockSpec((block, 12), lambda i: (i, 0)),
            pl.BlockSpec((block, 2), lambda i: (i, 0)),
        ] + [pl.BlockSpec(a.shape, lambda i: (0,) * a.ndim) for a in flat],
        out_specs=pl.BlockSpec((block, 12), lambda i: (i, 0)),
        out_shape=jax.ShapeDtypeStruct((N // 2, 12), jnp.float32),
    )(xbp.reshape(NP // 2, 16), u.reshape(N // 2, 12),
      aggr2p.reshape(2, NP // 2, 64), x.reshape(N // 2, 12),
      idf.reshape(N // 2, 2), *flat)
    return out.reshape(N, 6)


def kernel(x, ids, edge_index, u, alpha_params, alpha_inv_params, edge_params,
           node_params):
    idf = ids.astype(jnp.float32)
    sender = edge_index[0]
    receiver = edge_index[1]

    xbp = _prep(x, idf, alpha_params)

    send_p = jnp.concatenate([sender, jnp.zeros((EP - E,), jnp.int32)])
    recv_p = jnp.concatenate([receiver, jnp.full((EP - E,), N, jnp.int32)])
    recv_out = recv_p.reshape(-1, _EB16, 4, 4).transpose(0, 2, 1, 3).reshape(-1)
    recv3d = recv_out.reshape(EP // _SCH, _SCH // 128, 128)

    g2 = _sc_gather(xbp, send_p, recv_p)

    eo2 = _edge_mlp(g2, edge_params)

    zspan = jnp.zeros((_SPAN, 32), jnp.float32)
    aggr2p = _sc_scatter(eo2, recv3d, zspan)

    return _node_mlp(xbp, u, aggr2p, x, idf, node_params, alpha_inv_params)
